# Initial kernel scaffold; baseline (speedup 1.0000x reference)
import jax
import jax.numpy as jnp
from jax import lax
from jax.experimental import pallas as pl
from jax.experimental.pallas import tpu as pltpu
from jax.experimental.pallas import tpu_sc as plsc

N = 10000
E = 320000
D = 128
NC, NS = 2, 16
NW = NC * NS
CH = 128
NCHUNK = 79
EPW = NCHUNK * CH
EPAD = NW * EPW
NPAD = 10016
RPT = NPAD // NS
DEGW = 16
BLK = 1000

_HIGH = lax.Precision.HIGHEST
_mesh = plsc.VectorSubcoreMesh(core_axis_name="c", subcore_axis_name="s")


def _sc_segsum(with_deg):
    def body(y_hbm, srcp_hbm, dstp_hbm, zf_hbm, zd_hbm, ones_hbm,
             agg_hbm, deg_hbm, sidx, didx, rows, ones_v, acc, dacc, sem):
        c = lax.axis_index("c")
        s = lax.axis_index("s")
        w = c * NS + s
        r0 = s * RPT
        pltpu.sync_copy(zf_hbm.at[pl.ds(r0, RPT)], acc.at[pl.ds(r0, RPT)])
        if with_deg:
            pltpu.sync_copy(zd_hbm.at[pl.ds(r0, RPT)], dacc.at[pl.ds(r0, RPT)])
            pltpu.sync_copy(ones_hbm, ones_v)
        pltpu.sync_copy(srcp_hbm.at[w], sidx)
        pltpu.sync_copy(dstp_hbm.at[w], didx)
        plsc.subcore_barrier()

        def step(j, carry):
            pltpu.async_copy(y_hbm.at[sidx.at[j]], rows, sem).wait()
            pltpu.sync_copy(rows, acc.at[didx.at[j]], add=True)
            if with_deg:
                pltpu.sync_copy(ones_v, dacc.at[didx.at[j]], add=True)
            return carry

        lax.fori_loop(0, NCHUNK, step, 0)
        plsc.subcore_barrier()
        pltpu.sync_copy(acc.at[pl.ds(r0, RPT)], agg_hbm.at[c, pl.ds(r0, RPT)])
        if with_deg:
            pltpu.sync_copy(dacc.at[pl.ds(r0, RPT)], deg_hbm.at[c, pl.ds(r0, RPT)])

    if not with_deg:
        def body_nodeg(y_hbm, srcp_hbm, dstp_hbm, zf_hbm,
                       agg_hbm, sidx, didx, rows, acc, sem):
            body(y_hbm, srcp_hbm, dstp_hbm, zf_hbm, None, None,
                 agg_hbm, None, sidx, didx, rows, None, acc, None, sem)

        return pl.kernel(
            body_nodeg,
            out_type=jax.ShapeDtypeStruct((NC, NPAD, D), jnp.float32),
            mesh=_mesh,
            scratch_types=(
                pltpu.VMEM((NCHUNK, CH), jnp.int32),
                pltpu.VMEM((NCHUNK, CH), jnp.int32),
                pltpu.VMEM((CH, D), jnp.float32),
                pltpu.VMEM_SHARED((NPAD, D), jnp.float32),
                pltpu.SemaphoreType.DMA,
            ),
        )

    return pl.kernel(
        body,
        out_type=(jax.ShapeDtypeStruct((NC, NPAD, D), jnp.float32),
                  jax.ShapeDtypeStruct((NC, NPAD, DEGW), jnp.float32)),
        mesh=_mesh,
        scratch_types=(
            pltpu.VMEM((NCHUNK, CH), jnp.int32),
            pltpu.VMEM((NCHUNK, CH), jnp.int32),
            pltpu.VMEM((CH, D), jnp.float32),
            pltpu.VMEM((CH, DEGW), jnp.float32),
            pltpu.VMEM_SHARED((NPAD, D), jnp.float32),
            pltpu.VMEM_SHARED((NPAD, DEGW), jnp.float32),
            pltpu.SemaphoreType.DMA,
        ),
    )


def _dot(a, b):
    return jnp.dot(a, b, preferred_element_type=jnp.float32, precision=_HIGH)


def _k1_body(x_ref, w_ref, o_ref):
    o_ref[...] = _dot(x_ref[...], w_ref[...])


def _k1(x, wl1):
    return pl.pallas_call(
        _k1_body,
        grid=(N // BLK,),
        in_specs=[pl.BlockSpec((BLK, D), lambda i: (i, 0)),
                  pl.BlockSpec((D, D), lambda i: (0, 0))],
        out_specs=pl.BlockSpec((BLK, D), lambda i: (i, 0)),
        out_shape=jax.ShapeDtypeStruct((N, D), jnp.float32),
    )(x, wl1)


def _k2_body(x_ref, a0, a1, d0, d1, wr1_ref, bl1_ref, wl2_ref, wr2_ref,
             bl2_ref, y2_ref, r2_ref):
    agg = a0[0] + a1[0]
    deg = d0[0] + d1[0]
    degc = jnp.maximum(deg[:, 0:1], 1.0)
    h = agg / degc + _dot(x_ref[...], wr1_ref[...]) + bl1_ref[...]
    h = jnp.maximum(h, 0.0)
    y2_ref[...] = _dot(h, wl2_ref[...])
    r2_ref[...] = _dot(h, wr2_ref[...]) + bl2_ref[...]


def _k2(x, aggp, degp, wr1, bl1r, wl2, wr2, bl2r):
    part = lambda p: pl.BlockSpec((1, BLK, D), lambda i, p=p: (p, i, 0))
    dpart = lambda p: pl.BlockSpec((1, BLK, DEGW), lambda i, p=p: (p, i, 0))
    wfull = pl.BlockSpec((D, D), lambda i: (0, 0))
    bfull = pl.BlockSpec((1, D), lambda i: (0, 0))
    return pl.pallas_call(
        _k2_body,
        grid=(N // BLK,),
        in_specs=[pl.BlockSpec((BLK, D), lambda i: (i, 0)),
                  part(0), part(1), dpart(0), dpart(1),
                  wfull, bfull, wfull, wfull, bfull],
        out_specs=[pl.BlockSpec((BLK, D), lambda i: (i, 0)),
                   pl.BlockSpec((BLK, D), lambda i: (i, 0))],
        out_shape=[jax.ShapeDtypeStruct((N, D), jnp.float32),
                   jax.ShapeDtypeStruct((N, D), jnp.float32)],
    )(x, aggp, aggp, degp, degp, wr1, bl1r, wl2, wr2, bl2r)


def _k3_body(a0, a1, d0, d1, r2_ref, o_ref):
    agg = a0[0] + a1[0]
    deg = d0[0] + d1[0]
    degc = jnp.maximum(deg[:, 0:1], 1.0)
    o_ref[...] = agg / degc + r2_ref[...]


def _k3(aggp, degp, r2):
    part = lambda p: pl.BlockSpec((1, BLK, D), lambda i, p=p: (p, i, 0))
    dpart = lambda p: pl.BlockSpec((1, BLK, DEGW), lambda i, p=p: (p, i, 0))
    return pl.pallas_call(
        _k3_body,
        grid=(N // BLK,),
        in_specs=[part(0), part(1), dpart(0), dpart(1),
                  pl.BlockSpec((BLK, D), lambda i: (i, 0))],
        out_specs=pl.BlockSpec((BLK, D), lambda i: (i, 0)),
        out_shape=jax.ShapeDtypeStruct((N, D), jnp.float32),
    )(aggp, aggp, degp, degp, r2)


def kernel(x, edge_index, Wl1, bl1, Wr1, Wl2, bl2, Wr2):
    src = edge_index[0].astype(jnp.int32)
    dst = edge_index[1].astype(jnp.int32)
    pad = EPAD - E
    srcp = jnp.concatenate([src, jnp.zeros((pad,), jnp.int32)]).reshape(
        NW, NCHUNK, CH)
    dstp = jnp.concatenate([dst, jnp.full((pad,), N, jnp.int32)]).reshape(
        NW, NCHUNK, CH)
    zfeat = jnp.zeros((NPAD, D), jnp.float32)
    zdeg = jnp.zeros((NPAD, DEGW), jnp.float32)
    ones = jnp.ones((CH, DEGW), jnp.float32)
    bl1r = bl1.reshape(1, D)
    bl2r = bl2.reshape(1, D)

    y1 = _k1(x, Wl1)
    aggp1, degp = _sc_segsum(True)(y1, srcp, dstp, zfeat, zdeg, ones)
    y2, r2 = _k2(x, aggp1, degp, Wr1, bl1r, Wl2, Wr2, bl2r)
    aggp2 = _sc_segsum(False)(y2, srcp, dstp, zfeat)
    return _k3(aggp2, degp, r2)


# speedup vs baseline: 3.1043x; 3.1043x over previous
import jax
import jax.numpy as jnp
from jax import lax
from jax.experimental import pallas as pl
from jax.experimental.pallas import tpu as pltpu
from jax.experimental.pallas import tpu_sc as plsc

N = 10000
E = 320000
D = 128
NC, NS = 2, 16
NW = NC * NS
CH = 128
NCHUNK = 80
GS = 16
EPW = NCHUNK * CH
EPAD = NW * EPW
NPAD = 10240
RPT = NPAD // NS
BLK = 1000
DEGW = 16

_HIGH = lax.Precision.HIGHEST


def _mesh():
    return plsc.VectorSubcoreMesh(core_axis_name="c", subcore_axis_name="s",
                                  num_cores=NC, num_subcores=NS)


def _sc_segsum(with_deg):
    def body(y_hbm, srcp_hbm, dstp_hbm, zf_hbm, ones_hbm,
             agg_hbm, deg_hbm, sidx, didx, rows, acc, sem):
        c = lax.axis_index("c")
        s = lax.axis_index("s")
        w = c * NS + s
        r0 = s * RPT
        pltpu.sync_copy(zf_hbm.at[pl.ds(r0, RPT)], acc.at[pl.ds(r0, RPT)])
        plsc.subcore_barrier()

        def group(g, carry):
            pltpu.sync_copy(srcp_hbm.at[w, pl.ds(g * GS, GS)], sidx)
            pltpu.sync_copy(dstp_hbm.at[w, pl.ds(g * GS, GS)], didx)

            def step(j, c2):
                pltpu.async_copy(y_hbm.at[sidx.at[j]], rows, sem).wait()
                pltpu.sync_copy(rows, acc.at[didx.at[j]], add=True)
                return c2

            return lax.fori_loop(0, GS, step, carry)

        lax.fori_loop(0, NCHUNK // GS, group, 0)
        plsc.subcore_barrier()
        pltpu.sync_copy(acc.at[pl.ds(r0, RPT)], agg_hbm.at[c, pl.ds(r0, RPT)])

        if with_deg:
            plsc.subcore_barrier()
            pltpu.sync_copy(zf_hbm.at[pl.ds(r0, RPT)], acc.at[pl.ds(r0, RPT)])
            pltpu.sync_copy(ones_hbm, rows)
            plsc.subcore_barrier()

            def dgroup(g, carry):
                pltpu.sync_copy(dstp_hbm.at[w, pl.ds(g * GS, GS)], didx)

                def dstep(j, c2):
                    pltpu.sync_copy(rows, acc.at[didx.at[j]], add=True)
                    return c2

                return lax.fori_loop(0, GS, dstep, carry)

            lax.fori_loop(0, NCHUNK // GS, dgroup, 0)
            plsc.subcore_barrier()
            pltpu.sync_copy(acc.at[pl.ds(r0, RPT)],
                            deg_hbm.at[c, pl.ds(r0, RPT)])

    if not with_deg:
        def body_nodeg(y_hbm, srcp_hbm, dstp_hbm, zf_hbm,
                       agg_hbm, sidx, didx, rows, acc, sem):
            body(y_hbm, srcp_hbm, dstp_hbm, zf_hbm, None, agg_hbm, None,
                 sidx, didx, rows, acc, sem)

        return pl.kernel(
            body_nodeg,
            out_type=jax.ShapeDtypeStruct((NC, NPAD, D), jnp.float32),
            mesh=_mesh(),
            scratch_types=(
                pltpu.VMEM((GS, CH), jnp.int32),
                pltpu.VMEM((GS, CH), jnp.int32),
                pltpu.VMEM((CH, D), jnp.float32),
                pltpu.VMEM_SHARED((NPAD, D), jnp.float32),
                pltpu.SemaphoreType.DMA,
            ),
        )

    return pl.kernel(
        body,
        out_type=(jax.ShapeDtypeStruct((NC, NPAD, D), jnp.float32),
                  jax.ShapeDtypeStruct((NC, NPAD, D), jnp.float32)),
        mesh=_mesh(),
        scratch_types=(
            pltpu.VMEM((GS, CH), jnp.int32),
            pltpu.VMEM((GS, CH), jnp.int32),
            pltpu.VMEM((CH, D), jnp.float32),
            pltpu.VMEM_SHARED((NPAD, D), jnp.float32),
            pltpu.SemaphoreType.DMA,
        ),
    )


def _dot(a, b):
    return jnp.dot(a, b, preferred_element_type=jnp.float32, precision=_HIGH)


def _k1_body(x_ref, w_ref, o_ref):
    o_ref[...] = _dot(x_ref[...], w_ref[...])


def _k1(x, wl1):
    return pl.pallas_call(
        _k1_body,
        grid=(N // BLK,),
        in_specs=[pl.BlockSpec((BLK, D), lambda i: (i, 0)),
                  pl.BlockSpec((D, D), lambda i: (0, 0))],
        out_specs=pl.BlockSpec((BLK, D), lambda i: (i, 0)),
        out_shape=jax.ShapeDtypeStruct((N, D), jnp.float32),
    )(x, wl1)


def _k2_body(x_ref, a0, a1, d0, d1, wr1_ref, bl1_ref, wl2_ref, wr2_ref,
             bl2_ref, y2_ref, r2_ref, rd_ref):
    agg = a0[0] + a1[0]
    deg = d0[0][:, 0:1] + d1[0][:, 0:1]
    rdeg = 1.0 / jnp.maximum(deg, 1.0)
    h = agg * rdeg + _dot(x_ref[...], wr1_ref[...]) + bl1_ref[...]
    h = jnp.maximum(h, 0.0)
    y2_ref[...] = _dot(h, wl2_ref[...])
    r2_ref[...] = _dot(h, wr2_ref[...]) + bl2_ref[...]
    rd_ref[...] = jnp.broadcast_to(rdeg, (BLK, DEGW))


def _k2(x, aggp, degp, wr1, bl1r, wl2, wr2, bl2r):
    part = lambda a, p: pl.BlockSpec((1, BLK, D), lambda i, p=p: (p, i, 0))
    wfull = pl.BlockSpec((D, D), lambda i: (0, 0))
    bfull = pl.BlockSpec((1, D), lambda i: (0, 0))
    return pl.pallas_call(
        _k2_body,
        grid=(N // BLK,),
        in_specs=[pl.BlockSpec((BLK, D), lambda i: (i, 0)),
                  part(aggp, 0), part(aggp, 1), part(degp, 0), part(degp, 1),
                  wfull, bfull, wfull, wfull, bfull],
        out_specs=[pl.BlockSpec((BLK, D), lambda i: (i, 0)),
                   pl.BlockSpec((BLK, D), lambda i: (i, 0)),
                   pl.BlockSpec((BLK, DEGW), lambda i: (i, 0))],
        out_shape=[jax.ShapeDtypeStruct((N, D), jnp.float32),
                   jax.ShapeDtypeStruct((N, D), jnp.float32),
                   jax.ShapeDtypeStruct((N, DEGW), jnp.float32)],
    )(x, aggp, aggp, degp, degp, wr1, bl1r, wl2, wr2, bl2r)


def _k3_body(a0, a1, rd_ref, r2_ref, o_ref):
    agg = a0[0] + a1[0]
    o_ref[...] = agg * rd_ref[:, 0:1] + r2_ref[...]


def _k3(aggp, rdeg16, r2):
    part = lambda p: pl.BlockSpec((1, BLK, D), lambda i, p=p: (p, i, 0))
    return pl.pallas_call(
        _k3_body,
        grid=(N // BLK,),
        in_specs=[part(0), part(1),
                  pl.BlockSpec((BLK, DEGW), lambda i: (i, 0)),
                  pl.BlockSpec((BLK, D), lambda i: (i, 0))],
        out_specs=pl.BlockSpec((BLK, D), lambda i: (i, 0)),
        out_shape=jax.ShapeDtypeStruct((N, D), jnp.float32),
    )(aggp, aggp, rdeg16, r2)


def kernel(x, edge_index, Wl1, bl1, Wr1, Wl2, bl2, Wr2):
    src = edge_index[0].astype(jnp.int32)
    dst = edge_index[1].astype(jnp.int32)
    pad = EPAD - E
    srcp = jnp.concatenate([src, jnp.zeros((pad,), jnp.int32)]).reshape(
        NW, NCHUNK, CH)
    dstp = jnp.concatenate([dst, jnp.full((pad,), N, jnp.int32)]).reshape(
        NW, NCHUNK, CH)
    zfeat = jnp.zeros((NPAD, D), jnp.float32)
    ones128 = jnp.ones((CH, D), jnp.float32)
    bl1r = bl1.reshape(1, D)
    bl2r = bl2.reshape(1, D)

    y1 = _k1(x, Wl1)
    aggp1, degp = _sc_segsum(True)(y1, srcp, dstp, zfeat, ones128)
    y2, r2, rdeg16 = _k2(x, aggp1, degp, Wr1, bl1r, Wl2, Wr2, bl2r)
    aggp2 = _sc_segsum(False)(y2, srcp, dstp, zfeat)
    return _k3(aggp2, rdeg16, r2)


# speedup vs baseline: 3.2758x; 1.0552x over previous
import jax
import jax.numpy as jnp
from jax import lax
from jax.experimental import pallas as pl
from jax.experimental.pallas import tpu as pltpu
from jax.experimental.pallas import tpu_sc as plsc

N = 10000
E = 320000
D = 128
NC, NS = 2, 16
NW = NC * NS
CH = 128
NCHUNK = 80
GS = 8
EPW = NCHUNK * CH
EPAD = NW * EPW
NPAD = 10112
RPT = NPAD // NS
BLK = 1000
DEGW = 16

_HIGH = lax.Precision.HIGHEST


def _mesh():
    return plsc.VectorSubcoreMesh(core_axis_name="c", subcore_axis_name="s",
                                  num_cores=NC, num_subcores=NS)


def _sc_segsum(with_deg):
    def body(y_hbm, srcp_hbm, dstp_hbm, zf_hbm, ones_hbm,
             agg_hbm, deg_hbm, sidx, didx, rows, acc, sem0, sem1, dsem):
        c = lax.axis_index("c")
        s = lax.axis_index("s")
        w = c * NS + s
        r0 = s * RPT
        sems = (sem0, sem1)
        pltpu.sync_copy(zf_hbm.at[pl.ds(r0, RPT)], acc.at[pl.ds(r0, RPT)])
        plsc.subcore_barrier()

        def group(g, carry):
            pltpu.sync_copy(srcp_hbm.at[w, pl.ds(g * GS, GS)], sidx)
            pltpu.sync_copy(dstp_hbm.at[w, pl.ds(g * GS, GS)], didx)
            g0 = pltpu.async_copy(y_hbm.at[sidx.at[0]], rows.at[0], sems[0])
            g1 = pltpu.async_copy(y_hbm.at[sidx.at[1]], rows.at[1], sems[1])
            pend = [g0, g1]
            for j in range(GS):
                b = j % 2
                pend[b].wait()
                pltpu.sync_copy(rows.at[b], acc.at[didx.at[j]], add=True)
                if j + 2 < GS:
                    pend[b] = pltpu.async_copy(
                        y_hbm.at[sidx.at[j + 2]], rows.at[b], sems[b])
            return carry

        lax.fori_loop(0, NCHUNK // GS, group, 0)
        plsc.subcore_barrier()
        pltpu.sync_copy(acc.at[pl.ds(r0, RPT)], agg_hbm.at[c, pl.ds(r0, RPT)])

        if with_deg:
            plsc.subcore_barrier()
            pltpu.sync_copy(zf_hbm.at[pl.ds(r0, RPT)], acc.at[pl.ds(r0, RPT)])
            pltpu.sync_copy(ones_hbm, rows.at[0])
            plsc.subcore_barrier()

            def dgroup(g, carry):
                pltpu.sync_copy(dstp_hbm.at[w, pl.ds(g * GS, GS)], didx)
                descs = [pltpu.async_copy(rows.at[0], acc.at[didx.at[j]],
                                          dsem, add=True)
                         for j in range(GS)]
                for d in descs:
                    d.wait()
                return carry

            lax.fori_loop(0, NCHUNK // GS, dgroup, 0)
            plsc.subcore_barrier()
            pltpu.sync_copy(acc.at[pl.ds(r0, RPT)],
                            deg_hbm.at[c, pl.ds(r0, RPT)])

    if not with_deg:
        def body_nodeg(y_hbm, srcp_hbm, dstp_hbm, zf_hbm,
                       agg_hbm, sidx, didx, rows, acc, sem0, sem1):
            body(y_hbm, srcp_hbm, dstp_hbm, zf_hbm, None, agg_hbm, None,
                 sidx, didx, rows, acc, sem0, sem1, None)

        return pl.kernel(
            body_nodeg,
            out_type=jax.ShapeDtypeStruct((NC, NPAD, D), jnp.float32),
            mesh=_mesh(),
            scratch_types=(
                pltpu.VMEM((GS, CH), jnp.int32),
                pltpu.VMEM((GS, CH), jnp.int32),
                pltpu.VMEM((2, CH, D), jnp.float32),
                pltpu.VMEM_SHARED((NPAD, D), jnp.float32),
                pltpu.SemaphoreType.DMA,
                pltpu.SemaphoreType.DMA,
            ),
        )

    return pl.kernel(
        body,
        out_type=(jax.ShapeDtypeStruct((NC, NPAD, D), jnp.float32),
                  jax.ShapeDtypeStruct((NC, NPAD, D), jnp.float32)),
        mesh=_mesh(),
        scratch_types=(
            pltpu.VMEM((GS, CH), jnp.int32),
            pltpu.VMEM((GS, CH), jnp.int32),
            pltpu.VMEM((2, CH, D), jnp.float32),
            pltpu.VMEM_SHARED((NPAD, D), jnp.float32),
            pltpu.SemaphoreType.DMA,
            pltpu.SemaphoreType.DMA,
            pltpu.SemaphoreType.DMA,
        ),
    )


def _dot(a, b):
    return jnp.dot(a, b, preferred_element_type=jnp.float32, precision=_HIGH)


def _k1_body(x_ref, w_ref, o_ref):
    o_ref[...] = _dot(x_ref[...], w_ref[...])


def _k1(x, wl1):
    return pl.pallas_call(
        _k1_body,
        grid=(N // BLK,),
        in_specs=[pl.BlockSpec((BLK, D), lambda i: (i, 0)),
                  pl.BlockSpec((D, D), lambda i: (0, 0))],
        out_specs=pl.BlockSpec((BLK, D), lambda i: (i, 0)),
        out_shape=jax.ShapeDtypeStruct((N, D), jnp.float32),
    )(x, wl1)


def _k2_body(x_ref, a0, a1, d0, d1, wr1_ref, bl1_ref, wl2_ref, wr2_ref,
             bl2_ref, y2_ref, r2_ref, rd_ref):
    agg = a0[0] + a1[0]
    deg = d0[0][:, 0:1] + d1[0][:, 0:1]
    rdeg = 1.0 / jnp.maximum(deg, 1.0)
    h = agg * rdeg + _dot(x_ref[...], wr1_ref[...]) + bl1_ref[...]
    h = jnp.maximum(h, 0.0)
    y2_ref[...] = _dot(h, wl2_ref[...])
    r2_ref[...] = _dot(h, wr2_ref[...]) + bl2_ref[...]
    rd_ref[...] = jnp.broadcast_to(rdeg, (BLK, DEGW))


def _k2(x, aggp, degp, wr1, bl1r, wl2, wr2, bl2r):
    part = lambda a, p: pl.BlockSpec((1, BLK, D), lambda i, p=p: (p, i, 0))
    wfull = pl.BlockSpec((D, D), lambda i: (0, 0))
    bfull = pl.BlockSpec((1, D), lambda i: (0, 0))
    return pl.pallas_call(
        _k2_body,
        grid=(N // BLK,),
        in_specs=[pl.BlockSpec((BLK, D), lambda i: (i, 0)),
                  part(aggp, 0), part(aggp, 1), part(degp, 0), part(degp, 1),
                  wfull, bfull, wfull, wfull, bfull],
        out_specs=[pl.BlockSpec((BLK, D), lambda i: (i, 0)),
                   pl.BlockSpec((BLK, D), lambda i: (i, 0)),
                   pl.BlockSpec((BLK, DEGW), lambda i: (i, 0))],
        out_shape=[jax.ShapeDtypeStruct((N, D), jnp.float32),
                   jax.ShapeDtypeStruct((N, D), jnp.float32),
                   jax.ShapeDtypeStruct((N, DEGW), jnp.float32)],
    )(x, aggp, aggp, degp, degp, wr1, bl1r, wl2, wr2, bl2r)


def _k3_body(a0, a1, rd_ref, r2_ref, o_ref):
    agg = a0[0] + a1[0]
    o_ref[...] = agg * rd_ref[:, 0:1] + r2_ref[...]


def _k3(aggp, rdeg16, r2):
    part = lambda p: pl.BlockSpec((1, BLK, D), lambda i, p=p: (p, i, 0))
    return pl.pallas_call(
        _k3_body,
        grid=(N // BLK,),
        in_specs=[part(0), part(1),
                  pl.BlockSpec((BLK, DEGW), lambda i: (i, 0)),
                  pl.BlockSpec((BLK, D), lambda i: (i, 0))],
        out_specs=pl.BlockSpec((BLK, D), lambda i: (i, 0)),
        out_shape=jax.ShapeDtypeStruct((N, D), jnp.float32),
    )(aggp, aggp, rdeg16, r2)


def kernel(x, edge_index, Wl1, bl1, Wr1, Wl2, bl2, Wr2):
    src = edge_index[0].astype(jnp.int32)
    dst = edge_index[1].astype(jnp.int32)
    pad = EPAD - E
    srcp = jnp.concatenate([src, jnp.zeros((pad,), jnp.int32)]).reshape(
        NW, NCHUNK, CH)
    dstp = jnp.concatenate([dst, jnp.full((pad,), N, jnp.int32)]).reshape(
        NW, NCHUNK, CH)
    zfeat = jnp.zeros((NPAD, D), jnp.float32)
    ones128 = jnp.ones((CH, D), jnp.float32)
    bl1r = bl1.reshape(1, D)
    bl2r = bl2.reshape(1, D)

    y1 = _k1(x, Wl1)
    aggp1, degp = _sc_segsum(True)(y1, srcp, dstp, zfeat, ones128)
    y2, r2, rdeg16 = _k2(x, aggp1, degp, Wr1, bl1r, Wl2, Wr2, bl2r)
    aggp2 = _sc_segsum(False)(y2, srcp, dstp, zfeat)
    return _k3(aggp2, rdeg16, r2)
